# baseline (speedup 1.0000x reference)
import functools

import jax
import jax.numpy as jnp
from jax import lax
from jax.experimental import pallas as pl
from jax.experimental.pallas import tpu as pltpu
from jax.experimental.pallas import tpu_sc as plsc

NUM_TOKENS = 1000000
E_DIM = 64
BATCH = 4096
HIST = 200

NC = 2
NS = 16
NW = NC * NS
BBLK = BATCH // NW
NBUF = 4
CTILE = BATCH // 128

_MESH = plsc.VectorSubcoreMesh(core_axis_name="c", subcore_axis_name="s")


@functools.partial(
    pl.kernel,
    mesh=_MESH,
    out_type=jax.ShapeDtypeStruct((HIST, E_DIM // 8, CTILE, 1024), jnp.float32),
    scratch_types=[
        pltpu.VMEM((HIST, BBLK), jnp.int32),
        pltpu.VMEM((NBUF, BBLK, E_DIM), jnp.float32),
        pltpu.VMEM((NBUF, E_DIM * BBLK), jnp.float32),
        pltpu.SemaphoreType.DMA((NBUF,)),
        pltpu.SemaphoreType.DMA((NBUF,)),
    ],
    compiler_params=pltpu.CompilerParams(
        use_tc_tiling_on_sc=False, needs_layout_passes=False
    ),
)
def _emb_gather(xt2_hbm, table_hbm, out_hbm, idx_v, rows_v, tr_v, gsem, wsem):
    wid = lax.axis_index("s") * NC + lax.axis_index("c")
    b0 = wid * BBLK

    pltpu.sync_copy(xt2_hbm.at[:, pl.ds(b0, BBLK)], idx_v)

    def start_gather(l, s):
        pltpu.async_copy(table_hbm.at[idx_v.at[l]], rows_v.at[s], gsem.at[s])

    def wait_gather(l, s):
        pltpu.make_async_copy(
            table_hbm.at[idx_v.at[l]], rows_v.at[s], gsem.at[s]
        ).wait()

    def start_write(l, s):
        for r in range(E_DIM // 8):
            pltpu.async_copy(
                tr_v.at[s, pl.ds(r * 1024, 1024)], out_hbm.at[l, r, wid], wsem.at[s]
            )

    def wait_write(l, s):
        for r in range(E_DIM // 8):
            pltpu.make_async_copy(
                tr_v.at[s, pl.ds(r * 1024, 1024)], out_hbm.at[l, r, wid], wsem.at[s]
            ).wait()

    ji = lax.iota(jnp.int32, 16)
    e128 = [(ee * 16 + ji) * 128 for ee in range(E_DIM // 16)]

    def transpose_block(s):
        @plsc.parallel_loop(0, BBLK, unroll=8)
        def jrow(j):
            jf = jnp.full((16,), j, jnp.int32)
            for ee in range(E_DIM // 16):
                strip = rows_v[s, j, pl.ds(ee * 16, 16)]
                plsc.store_scatter(tr_v.at[s], [e128[ee] + jf], strip)

    start_gather(0, 0)
    start_gather(1, 1)
    start_gather(2, 2)

    def body(i, carry):
        l = i * NBUF
        for s in range(NBUF):
            cur = l + s
            la = cur + 3

            @pl.when(la < HIST)
            def _():
                start_gather(la, (s + 3) % NBUF)

            wait_gather(cur, s)

            @pl.when(cur >= NBUF)
            def _():
                wait_write(cur - NBUF, s)

            transpose_block(s)
            start_write(cur, s)
        return carry

    lax.fori_loop(0, HIST // NBUF, body, 0)

    for s in range(NBUF):
        wait_write(HIST - NBUF + s, s)


def kernel(x, table):
    xt2 = x.T.astype(jnp.int32)
    tp = table
    outp = _emb_gather(xt2, tp)
    out5 = outp.reshape(HIST, E_DIM // 8, CTILE, 8, 128)
    return out5.transpose(2, 4, 0, 1, 3).reshape(BATCH, HIST, E_DIM)


# speedup vs baseline: 1.0529x; 1.0529x over previous
import functools

import jax
import jax.numpy as jnp
from jax import lax
from jax.experimental import pallas as pl
from jax.experimental.pallas import tpu as pltpu
from jax.experimental.pallas import tpu_sc as plsc

NUM_TOKENS = 1000000
E_DIM = 64
BATCH = 4096
HIST = 200

NC = 2
NS = 16
NW = NC * NS
BBLK = BATCH // NW
NBUF = 4
CTILE = BATCH // 128

_MESH = plsc.VectorSubcoreMesh(core_axis_name="c", subcore_axis_name="s")


@functools.partial(
    pl.kernel,
    mesh=_MESH,
    out_type=jax.ShapeDtypeStruct((HIST, E_DIM // 8, CTILE, 1024), jnp.float32),
    scratch_types=[
        pltpu.VMEM((HIST, BBLK), jnp.int32),
        pltpu.VMEM((NBUF, BBLK, E_DIM), jnp.float32),
        pltpu.VMEM((NBUF, E_DIM * BBLK), jnp.float32),
        pltpu.SemaphoreType.DMA((NBUF,)),
        pltpu.SemaphoreType.DMA((NBUF,)),
    ],
    compiler_params=pltpu.CompilerParams(
        use_tc_tiling_on_sc=False, needs_layout_passes=False
    ),
)
def _emb_gather(xt2_hbm, table_hbm, out_hbm, idx_v, rows_v, tr_v, gsem, wsem):
    wid = lax.axis_index("s") * NC + lax.axis_index("c")
    b0 = wid * BBLK

    pltpu.sync_copy(xt2_hbm.at[:, pl.ds(b0, BBLK)], idx_v)

    def start_gather(l, s):
        pltpu.async_copy(table_hbm.at[idx_v.at[l]], rows_v.at[s], gsem.at[s])

    def wait_gather(l, s):
        pltpu.make_async_copy(
            table_hbm.at[idx_v.at[l]], rows_v.at[s], gsem.at[s]
        ).wait()

    def start_write(l, s):
        for r in range(E_DIM // 8):
            pltpu.async_copy(
                tr_v.at[s, pl.ds(r * 1024, 1024)], out_hbm.at[l, r, wid], wsem.at[s]
            )

    def wait_write(l, s):
        for r in range(E_DIM // 8):
            pltpu.make_async_copy(
                tr_v.at[s, pl.ds(r * 1024, 1024)], out_hbm.at[l, r, wid], wsem.at[s]
            ).wait()

    ji = lax.iota(jnp.int32, 16)
    e128 = [(ee * 16 + ji) * 128 for ee in range(E_DIM // 16)]

    def transpose_block(s):
        @plsc.parallel_loop(0, BBLK, unroll=16)
        def jrow(j):
            jf = jnp.full((16,), j, jnp.int32)
            for ee in range(E_DIM // 16):
                strip = rows_v[s, j, pl.ds(ee * 16, 16)]
                plsc.store_scatter(tr_v.at[s], [e128[ee] + jf], strip)

    start_gather(0, 0)
    start_gather(1, 1)
    start_gather(2, 2)

    def body(i, carry):
        l = i * NBUF
        for s in range(NBUF):
            cur = l + s
            la = cur + 3

            @pl.when(la < HIST)
            def _():
                start_gather(la, (s + 3) % NBUF)

            wait_gather(cur, s)

            @pl.when(cur >= NBUF)
            def _():
                wait_write(cur - NBUF, s)

            transpose_block(s)
            start_write(cur, s)
        return carry

    lax.fori_loop(0, HIST // NBUF, body, 0)

    for s in range(NBUF):
        wait_write(HIST - NBUF + s, s)


def kernel(x, table):
    xt2 = x.T.astype(jnp.int32) * 2
    tp = jnp.pad(table, ((0, 0), (0, 64))).reshape(2 * NUM_TOKENS, E_DIM)
    outp = _emb_gather(xt2, tp)
    out5 = outp.reshape(HIST, E_DIM // 8, CTILE, 8, 128)
    return out5.transpose(2, 4, 0, 1, 3).reshape(BATCH, HIST, E_DIM)


# speedup vs baseline: 1.0569x; 1.0038x over previous
import functools

import jax
import jax.numpy as jnp
from jax import lax
from jax.experimental import pallas as pl
from jax.experimental.pallas import tpu as pltpu
from jax.experimental.pallas import tpu_sc as plsc

NUM_TOKENS = 1000000
E_DIM = 64
BATCH = 4096
HIST = 200

NC = 2
NS = 16
NW = NC * NS
BBLK = BATCH // NW
NBUF = 4
CTILE = BATCH // 128

_MESH = plsc.VectorSubcoreMesh(core_axis_name="c", subcore_axis_name="s")


@functools.partial(
    pl.kernel,
    mesh=_MESH,
    out_type=jax.ShapeDtypeStruct((HIST, E_DIM // 8, CTILE, 1024), jnp.float32),
    scratch_types=[
        pltpu.VMEM((HIST, BBLK), jnp.int32),
        pltpu.VMEM((NBUF, BBLK, E_DIM), jnp.float32),
        pltpu.VMEM((NBUF, E_DIM // 8, 8 * BBLK), jnp.float32),
        pltpu.SemaphoreType.DMA((NBUF,)),
        pltpu.SemaphoreType.DMA((NBUF,)),
    ],
    compiler_params=pltpu.CompilerParams(
        use_tc_tiling_on_sc=False, needs_layout_passes=False
    ),
)
def _emb_gather(xt2_hbm, table_hbm, out_hbm, idx_v, rows_v, tr_v, gsem, wsem):
    wid = lax.axis_index("s") * NC + lax.axis_index("c")
    b0 = wid * BBLK

    pltpu.sync_copy(xt2_hbm.at[:, pl.ds(b0, BBLK)], idx_v)

    def start_gather(l, s):
        pltpu.async_copy(table_hbm.at[idx_v.at[l]], rows_v.at[s], gsem.at[s])

    def wait_gather(l, s):
        pltpu.make_async_copy(
            table_hbm.at[idx_v.at[l]], rows_v.at[s], gsem.at[s]
        ).wait()

    def start_write(l, s):
        pltpu.async_copy(tr_v.at[s], out_hbm.at[l, :, wid], wsem.at[s])

    def wait_write(l, s):
        pltpu.make_async_copy(
            tr_v.at[s], out_hbm.at[l, :, wid], wsem.at[s]
        ).wait()

    ji = lax.iota(jnp.int32, 16)
    r16 = [(ee * 16 + ji) >> 3 for ee in range(E_DIM // 16)]
    f16 = [((ee * 16 + ji) & 7) * 128 for ee in range(E_DIM // 16)]

    def transpose_block(s):
        @plsc.parallel_loop(0, BBLK, unroll=16)
        def jrow(j):
            jf = jnp.full((16,), j, jnp.int32)
            for ee in range(E_DIM // 16):
                strip = rows_v[s, j, pl.ds(ee * 16, 16)]
                plsc.store_scatter(tr_v.at[s], [r16[ee], f16[ee] + jf], strip)

    start_gather(0, 0)
    start_gather(1, 1)
    start_gather(2, 2)

    def body(i, carry):
        l = i * NBUF
        for s in range(NBUF):
            cur = l + s
            la = cur + 3

            @pl.when(la < HIST)
            def _():
                start_gather(la, (s + 3) % NBUF)

            wait_gather(cur, s)

            @pl.when(cur >= NBUF)
            def _():
                wait_write(cur - NBUF, s)

            transpose_block(s)
            start_write(cur, s)
        return carry

    lax.fori_loop(0, HIST // NBUF, body, 0)

    for s in range(NBUF):
        wait_write(HIST - NBUF + s, s)


def kernel(x, table):
    xt2 = x.T.astype(jnp.int32) * 2
    tp = jnp.pad(table, ((0, 0), (0, 64))).reshape(2 * NUM_TOKENS, E_DIM)
    outp = _emb_gather(xt2, tp)
    out5 = outp.reshape(HIST, E_DIM // 8, CTILE, 8, 128)
    return out5.transpose(2, 4, 0, 1, 3).reshape(BATCH, HIST, E_DIM)
